# baseline (speedup 1.0000x reference)
import jax
import jax.numpy as jnp
from jax import lax
from jax.experimental import pallas as pl
from jax.experimental.pallas import tpu as pltpu
from jax.experimental.pallas import tpu_sc as plsc

_NC = 2
_NS = 16
_NW = _NC * _NS

_NB = 4096
_NT = 50
_D = 32
_BLK = _NB // _NW
_PER_W = _BLK * _NT

_V = 1000000
_SR = _V // 4
_NTILE = _V // 128
_TAIL = _V - _NTILE * 128
_PER_W_T = _NTILE // _NW
_EXTRA = _NTILE - _PER_W_T * _NW


def _relayout_body(tin_hbm, tail_hbm, out_hbm, ta, tb, oa, ob, lsem, wsem):
    wid = lax.axis_index("s") * _NC + lax.axis_index("c")
    lane = lax.iota(jnp.int32, 16)
    lane_lo = lane
    lane_hi = lane + 16

    _K = 4
    n_it = _PER_W_T // _K

    def load(m, tbuf, sem):
        for r in range(4):
            pltpu.async_copy(
                tin_hbm.at[pl.ds(8 * r, 8), pl.ds(m * 128, 128 * _K)],
                tbuf.at[pl.ds(8 * r, 8)], sem,
            )

    def transpose(tbuf, obuf, ngrp):
        zero = jnp.zeros((16,), jnp.int32)

        def body(i, carry):
            k = jax.lax.shift_right_logical(i, 4)
            s = jax.lax.bitwise_and(i, 15) * 2
            base = k * 128 + s * 4
            row = k * 32 + s
            vals = [
                plsc.load_gather(
                    tbuf,
                    [lane_lo if (g % 2) == 0 else lane_hi,
                     zero + base + (g % 8) // 2 + (g // 8) * 4],
                )
                for g in range(16)
            ]
            for g in range(8):
                obuf[row, pl.ds(g * 16, 16)] = vals[g]
            for g in range(8):
                obuf[row + 1, pl.ds(g * 16, 16)] = vals[8 + g]
            return carry

        lax.fori_loop(0, ngrp * 16, body, 0)

    def flip(m, tbuf, obuf, ls, ws, first):
        pltpu.make_async_copy(
            tin_hbm.at[:, pl.ds(0, 128 * _K)], tbuf, ls
        ).wait()
        if not first:
            pltpu.make_async_copy(
                obuf, out_hbm.at[pl.ds(0, 32 * _K)], ws
            ).wait()
        transpose(tbuf, obuf, _K)
        pltpu.async_copy(obuf, out_hbm.at[pl.ds(m * 32, 32 * _K)], ws)

    m0 = wid * _PER_W_T

    load(m0, ta, lsem.at[0])
    load(m0 + _K, tb, lsem.at[1])
    flip(m0, ta, oa, lsem.at[0], wsem.at[0], True)
    load(m0 + 2 * _K, ta, lsem.at[0])
    flip(m0 + _K, tb, ob, lsem.at[1], wsem.at[1], True)
    load(m0 + 3 * _K, tb, lsem.at[1])

    def step(h, carry):
        j = h * 2
        m = m0 + j * _K
        flip(m, ta, oa, lsem.at[0], wsem.at[0], False)

        @pl.when(j + 2 < n_it)
        def _():
            load(m + 2 * _K, ta, lsem.at[0])

        flip(m + _K, tb, ob, lsem.at[1], wsem.at[1], False)

        @pl.when(j + 3 < n_it)
        def _():
            load(m + 3 * _K, tb, lsem.at[1])

        return carry

    lax.fori_loop(1, n_it // 2, step, 0)

    flip(m0 + (n_it - 1) * _K, ta, oa, lsem.at[0], wsem.at[0], False)

    @pl.when(wid < _EXTRA)
    def _():
        m = _PER_W_T * _NW + wid
        for r in range(4):
            pltpu.async_copy(
                tin_hbm.at[pl.ds(8 * r, 8), pl.ds(m * 128, 128)],
                tb.at[pl.ds(8 * r, 8), pl.ds(0, 128)], lsem.at[1],
            )
        pltpu.make_async_copy(
            tin_hbm.at[:, pl.ds(0, 128)], tb.at[:, pl.ds(0, 128)], lsem.at[1]
        ).wait()
        pltpu.make_async_copy(
            ob, out_hbm.at[pl.ds(0, 32 * _K)], wsem.at[1]
        ).wait()
        transpose(tb, ob, 1)
        pltpu.async_copy(
            ob.at[pl.ds(0, 32)], out_hbm.at[pl.ds(m * 32, 32)], wsem.at[1]
        )
        pltpu.make_async_copy(
            ob.at[pl.ds(0, 32)], out_hbm.at[pl.ds(0, 32)], wsem.at[1]
        ).wait()

    pltpu.make_async_copy(
        oa, out_hbm.at[pl.ds(0, 32 * _K)], wsem.at[0]
    ).wait()

    @pl.when(wid == _EXTRA)
    def _():
        pltpu.sync_copy(tail_hbm, oa.at[pl.ds(96, 16)])
        pltpu.sync_copy(
            oa.at[pl.ds(96, 16)], out_hbm.at[pl.ds(_NTILE * 32, 16)]
        )

    @pl.when(wid >= _EXTRA)
    def _():
        pltpu.make_async_copy(
            ob, out_hbm.at[pl.ds(0, 32 * _K)], wsem.at[1]
        ).wait()


def _gather_body(table_hbm, idx_hbm, out_hbm,
                 idx_v, sidx_a, sidx_b, ov_a, ov_b, rows_a, rows_b,
                 tr_a, tr_b, gsem, wsem):
    wid = lax.axis_index("s") * _NC + lax.axis_index("c")
    b0 = wid * _BLK
    pltpu.sync_copy(idx_hbm.at[pl.ds(b0 * _NT, _PER_W)], idx_v)

    lane = lax.iota(jnp.int32, 16)
    stride_lane = lane * _NT

    def prep(t, sidx_v, ov_v):
        ivs = [
            plsc.load_gather(idx_v, [stride_lane + (g * 16 * _NT) + t])
            for g in range(_BLK // 16)
        ]
        for g, iv in enumerate(ivs):
            sidx_v[pl.ds(g * 16, 16)] = jax.lax.shift_right_logical(iv, 2)
        for g, iv in enumerate(ivs):
            ov_v[pl.ds(g * 16, 16)] = jax.lax.bitwise_and(iv, 3) * _D

    def fire(sidx_v, rows_v, sem):
        pltpu.async_copy(table_hbm.at[sidx_v], rows_v, sem)

    def extract(t, sidx_v, rows_v, ov_v, tr_v, gs, ws, first):
        pltpu.make_async_copy(table_hbm.at[sidx_v], rows_v, gs).wait()
        if not first:
            pltpu.make_async_copy(
                tr_v, out_hbm.at[t, :, pl.ds(b0, _BLK)], ws
            ).wait()
        for g in range(_BLK // 16):
            kvec = lane + g * 16
            ov = ov_v[pl.ds(g * 16, 16)]
            vals = [
                plsc.load_gather(rows_v, [kvec, ov + c]) for c in range(_D)
            ]
            for c in range(_D):
                tr_v[c, pl.ds(g * 16, 16)] = vals[c]
        pltpu.async_copy(tr_v, out_hbm.at[t, :, pl.ds(b0, _BLK)], ws)

    prep(0, sidx_a, ov_a)
    fire(sidx_a, rows_a, gsem.at[0])

    def step(m, carry):
        t0 = m * 2
        prep(t0 + 1, sidx_b, ov_b)
        fire(sidx_b, rows_b, gsem.at[1])
        extract(t0, sidx_a, rows_a, ov_a, tr_a, gsem.at[0], wsem.at[0], False)

        @pl.when(t0 + 2 < _NT)
        def _():
            prep(t0 + 2, sidx_a, ov_a)
            fire(sidx_a, rows_a, gsem.at[0])

        extract(t0 + 1, sidx_b, rows_b, ov_b, tr_b, gsem.at[1], wsem.at[1],
                False)
        return carry

    prep(1, sidx_b, ov_b)
    fire(sidx_b, rows_b, gsem.at[1])
    extract(0, sidx_a, rows_a, ov_a, tr_a, gsem.at[0], wsem.at[0], True)
    prep(2, sidx_a, ov_a)
    fire(sidx_a, rows_a, gsem.at[0])
    extract(1, sidx_b, rows_b, ov_b, tr_b, gsem.at[1], wsem.at[1], True)

    lax.fori_loop(1, _NT // 2, step, 0)

    pltpu.make_async_copy(
        tr_a, out_hbm.at[_NT - 2, :, pl.ds(b0, _BLK)], wsem.at[0]
    ).wait()
    pltpu.make_async_copy(
        tr_b, out_hbm.at[_NT - 1, :, pl.ds(b0, _BLK)], wsem.at[1]
    ).wait()


@jax.jit
def kernel(embeddings, inputs):
    table_t = embeddings.T
    tail2 = embeddings[_NTILE * 128:].reshape(16, 128)
    idxf = inputs.reshape(_NB * _NT)

    relayout = pl.kernel(
        _relayout_body,
        out_type=jax.ShapeDtypeStruct((_SR, 128), jnp.float32),
        mesh=plsc.VectorSubcoreMesh(core_axis_name="c", subcore_axis_name="s"),
        scratch_types=[
            pltpu.VMEM((32, 512), jnp.float32),
            pltpu.VMEM((32, 512), jnp.float32),
            pltpu.VMEM((128, 128), jnp.float32),
            pltpu.VMEM((128, 128), jnp.float32),
            pltpu.SemaphoreType.DMA((2,)),
            pltpu.SemaphoreType.DMA((2,)),
        ],
        compiler_params=pltpu.CompilerParams(
            use_tc_tiling_on_sc=True, needs_layout_passes=False
        ),
    )
    table2 = relayout(table_t, tail2)

    gather = pl.kernel(
        _gather_body,
        out_type=jax.ShapeDtypeStruct((_NT, _D, _NB), jnp.float32),
        mesh=plsc.VectorSubcoreMesh(core_axis_name="c", subcore_axis_name="s"),
        scratch_types=[
            pltpu.VMEM((_PER_W,), jnp.int32),
            pltpu.VMEM((_BLK,), jnp.int32),
            pltpu.VMEM((_BLK,), jnp.int32),
            pltpu.VMEM((_BLK,), jnp.int32),
            pltpu.VMEM((_BLK,), jnp.int32),
            pltpu.VMEM((_BLK, 128), jnp.float32),
            pltpu.VMEM((_BLK, 128), jnp.float32),
            pltpu.VMEM((_D, _BLK), jnp.float32),
            pltpu.VMEM((_D, _BLK), jnp.float32),
            pltpu.SemaphoreType.DMA((2,)),
            pltpu.SemaphoreType.DMA((2,)),
        ],
        compiler_params=pltpu.CompilerParams(
            use_tc_tiling_on_sc=True, needs_layout_passes=False
        ),
    )
    out_t = gather(table2, idxf)
    return out_t.transpose(2, 0, 1)


# speedup vs baseline: 1.0288x; 1.0288x over previous
import jax
import jax.numpy as jnp
from jax import lax
from jax.experimental import pallas as pl
from jax.experimental.pallas import tpu as pltpu
from jax.experimental.pallas import tpu_sc as plsc

_NC = 2
_NS = 16
_NW = _NC * _NS

_NB = 4096
_NT = 50
_D = 32
_BLK = _NB // _NW
_PER_W = _BLK * _NT


def _body(table_hbm, idx_hbm, out_hbm,
          idx_v, sidx_a, sidx_b, ov_a, ov_b, rows_a, rows_b, tr_a, tr_b,
          gsem, wsem):
    wid = lax.axis_index("s") * _NC + lax.axis_index("c")
    b0 = wid * _BLK
    pltpu.sync_copy(idx_hbm.at[pl.ds(b0 * _NT, _PER_W)], idx_v)

    lane = lax.iota(jnp.int32, 16)
    stride_lane = lane * _NT

    def prep(t, sidx_v, ov_v):
        ivs = [
            plsc.load_gather(idx_v, [stride_lane + (g * 16 * _NT) + t])
            for g in range(_BLK // 16)
        ]
        for g, iv in enumerate(ivs):
            sidx_v[pl.ds(g * 16, 16)] = jax.lax.shift_right_logical(iv, 2)
        for g, iv in enumerate(ivs):
            ov_v[pl.ds(g * 16, 16)] = jax.lax.bitwise_and(iv, 3) * _D

    def fire(sidx_v, rows_v, sem):
        pltpu.async_copy(table_hbm.at[sidx_v], rows_v, sem)

    def extract(t, sidx_v, rows_v, ov_v, tr_v, gs, ws, first):
        pltpu.make_async_copy(table_hbm.at[sidx_v], rows_v, gs).wait()
        if not first:
            pltpu.make_async_copy(
                tr_v, out_hbm.at[t, :, pl.ds(b0, _BLK)], ws
            ).wait()
        for g in range(_BLK // 16):
            kvec = lane + g * 16
            ov = ov_v[pl.ds(g * 16, 16)]
            vals = [
                plsc.load_gather(rows_v, [kvec, ov + c]) for c in range(_D)
            ]
            for c in range(_D):
                tr_v[c, pl.ds(g * 16, 16)] = vals[c]
        pltpu.async_copy(tr_v, out_hbm.at[t, :, pl.ds(b0, _BLK)], ws)

    prep(0, sidx_a, ov_a)
    fire(sidx_a, rows_a, gsem.at[0])

    def step(m, carry):
        t0 = m * 2

        prep(t0 + 1, sidx_b, ov_b)
        fire(sidx_b, rows_b, gsem.at[1])
        extract(t0, sidx_a, rows_a, ov_a, tr_a, gsem.at[0], wsem.at[0], False)

        @pl.when(t0 + 2 < _NT)
        def _():
            prep(t0 + 2, sidx_a, ov_a)
            fire(sidx_a, rows_a, gsem.at[0])

        extract(t0 + 1, sidx_b, rows_b, ov_b, tr_b, gsem.at[1], wsem.at[1],
                False)
        return carry

    prep(1, sidx_b, ov_b)
    fire(sidx_b, rows_b, gsem.at[1])
    extract(0, sidx_a, rows_a, ov_a, tr_a, gsem.at[0], wsem.at[0], True)
    prep(2, sidx_a, ov_a)
    fire(sidx_a, rows_a, gsem.at[0])
    extract(1, sidx_b, rows_b, ov_b, tr_b, gsem.at[1], wsem.at[1], True)

    lax.fori_loop(1, _NT // 2, step, 0)

    pltpu.make_async_copy(
        tr_a, out_hbm.at[_NT - 2, :, pl.ds(b0, _BLK)], wsem.at[0]
    ).wait()
    pltpu.make_async_copy(
        tr_b, out_hbm.at[_NT - 1, :, pl.ds(b0, _BLK)], wsem.at[1]
    ).wait()


@jax.jit
def kernel(embeddings, inputs):
    table2 = embeddings.reshape(250000, 128)
    idxf = inputs.reshape(_NB * _NT)
    run = pl.kernel(
        _body,
        out_type=jax.ShapeDtypeStruct((_NT, _D, _NB), jnp.float32),
        mesh=plsc.VectorSubcoreMesh(core_axis_name="c", subcore_axis_name="s"),
        scratch_types=[
            pltpu.VMEM((_PER_W,), jnp.int32),
            pltpu.VMEM((_BLK,), jnp.int32),
            pltpu.VMEM((_BLK,), jnp.int32),
            pltpu.VMEM((_BLK,), jnp.int32),
            pltpu.VMEM((_BLK,), jnp.int32),
            pltpu.VMEM((_BLK, 128), jnp.float32),
            pltpu.VMEM((_BLK, 128), jnp.float32),
            pltpu.VMEM((_D, _BLK), jnp.float32),
            pltpu.VMEM((_D, _BLK), jnp.float32),
            pltpu.SemaphoreType.DMA((2,)),
            pltpu.SemaphoreType.DMA((2,)),
        ],
        compiler_params=pltpu.CompilerParams(
            use_tc_tiling_on_sc=True, needs_layout_passes=False
        ),
    )
    out_t = run(table2, idxf)
    return out_t.transpose(2, 0, 1)
